# Initial kernel scaffold; baseline (speedup 1.0000x reference)
import jax
import jax.numpy as jnp
from jax.experimental import pallas as pl


def kernel(id_emb, llm_emb, W1, b1, W2, b2, Wg1, bg1, Wg2, bg2):
    raise NotImplementedError("write your pallas kernel here")


# speedup vs baseline: 1.0219x; 1.0219x over previous
import functools

import jax
import jax.numpy as jnp
from jax.experimental import pallas as pl
from jax.experimental.pallas import tpu as pltpu

ID_DIM = 128
LLM_DIM = 2048
OUT_DIM = 1024
E = 8
TOPK = 2
TOK = 2048
IN_DIM = ID_DIM + LLM_DIM
HID = 2 * OUT_DIM


def _gate_body(x_ref, wg1_ref, bg1_ref, wg2_ref, bg2_ref, wmat_ref):
    x = x_ref[...]
    h = jnp.maximum(
        jnp.dot(x, wg1_ref[...], preferred_element_type=jnp.float32)
        + bg1_ref[...],
        0.0,
    )
    logits = (
        jnp.dot(h, wg2_ref[...], preferred_element_type=jnp.float32)
        + bg2_ref[...]
    )
    iota = jax.lax.broadcasted_iota(jnp.int32, (TOK, E), 1)
    m1 = jnp.max(logits, axis=1, keepdims=True)
    idx1 = jnp.min(jnp.where(logits == m1, iota, E), axis=1, keepdims=True)
    masked = jnp.where(iota == idx1, -jnp.inf, logits)
    m2 = jnp.max(masked, axis=1, keepdims=True)
    idx2 = jnp.min(
        jnp.where((logits == m2) & (iota != idx1), iota, E),
        axis=1,
        keepdims=True,
    )
    p2 = 1.0 / (1.0 + jnp.exp(m1 - m2))
    p1 = 1.0 - p2
    wmat_ref[...] = jnp.where(iota == idx1, p1, 0.0) + jnp.where(
        iota == idx2, p2, 0.0
    )


def _moe_body(wmat_ref, x_ref, w1_ref, b1_ref, w2_ref, b2_ref, out_ref):
    e = pl.program_id(0)
    hh = pl.program_id(1)

    @pl.when((e == 0) & (hh == 0))
    def _init():
        out_ref[...] = jnp.zeros_like(out_ref)

    x = x_ref[...]
    h = jnp.maximum(
        jnp.dot(x, w1_ref[0], preferred_element_type=jnp.float32)
        + b1_ref[0, 0],
        0.0,
    )
    acc = jnp.dot(h, w2_ref[0], preferred_element_type=jnp.float32)
    iota = jax.lax.broadcasted_iota(jnp.int32, (TOK, E), 1)
    w_col = jnp.sum(
        jnp.where(iota == e, wmat_ref[...], 0.0), axis=1, keepdims=True
    )

    @pl.when(hh == 0)
    def _first_half():
        out_ref[...] += w_col * (acc + b2_ref[pl.ds(e, 1), :])

    @pl.when(hh == 1)
    def _second_half():
        out_ref[...] += w_col * acc


def kernel(id_emb, llm_emb, W1, b1, W2, b2, Wg1, bg1, Wg2, bg2):
    combined = jnp.concatenate([id_emb, llm_emb], axis=-1)

    wmat = pl.pallas_call(
        _gate_body,
        out_shape=jax.ShapeDtypeStruct((TOK, E), jnp.float32),
        in_specs=[
            pl.BlockSpec((TOK, IN_DIM), lambda: (0, 0)),
            pl.BlockSpec((IN_DIM, 2 * E), lambda: (0, 0)),
            pl.BlockSpec((2 * E,), lambda: (0,)),
            pl.BlockSpec((2 * E, E), lambda: (0, 0)),
            pl.BlockSpec((E,), lambda: (0,)),
        ],
        out_specs=pl.BlockSpec((TOK, E), lambda: (0, 0)),
    )(combined, Wg1, bg1, Wg2, bg2)

    HH = HID // 2
    b1r = b1.reshape(E, 2, 1, HH)
    out = pl.pallas_call(
        _moe_body,
        grid=(E, 2),
        out_shape=jax.ShapeDtypeStruct((TOK, OUT_DIM), jnp.float32),
        in_specs=[
            pl.BlockSpec((TOK, E), lambda e, hh: (0, 0)),
            pl.BlockSpec((TOK, IN_DIM), lambda e, hh: (0, 0)),
            pl.BlockSpec((1, IN_DIM, HH), lambda e, hh: (e, 0, hh)),
            pl.BlockSpec((1, 1, 1, HH), lambda e, hh: (e, hh, 0, 0)),
            pl.BlockSpec((1, HH, OUT_DIM), lambda e, hh: (e, hh, 0)),
            pl.BlockSpec((E, OUT_DIM), lambda e, hh: (0, 0)),
        ],
        out_specs=pl.BlockSpec((TOK, OUT_DIM), lambda e, hh: (0, 0)),
        compiler_params=pltpu.CompilerParams(
            vmem_limit_bytes=100 * 1024 * 1024
        ),
    )(wmat, combined, W1, b1r, W2, b2)
    return out
